# baseline (speedup 1.0000x reference)
import jax
import jax.numpy as jnp
from jax.experimental import pallas as pl
from jax.experimental.pallas import tpu as pltpu

_CH = 5
_NB = 10


def _posadd_kernel(pos_ref, x_hbm, o_hbm, ibuf, obuf, isem, osem):
    n = x_hbm.shape[0] // _CH
    nsup = n // _NB

    def icopy(i, slot):
        return pltpu.make_async_copy(
            x_hbm.at[pl.ds(i * _CH, _CH)], ibuf.at[slot], isem.at[slot]
        )

    def ocopy(i, slot):
        return pltpu.make_async_copy(
            obuf.at[slot], o_hbm.at[pl.ds(i * _CH, _CH)], osem.at[slot]
        )

    for s in range(_NB):
        icopy(s, s).start()

    def super_body(t, carry):
        for s in range(_NB):
            i = t * _NB + s
            icopy(i, s).wait()

            @pl.when(t > 0)
            def _():
                ocopy(i - _NB, s).wait()

            pos = pos_ref[pl.ds(i * _CH, _CH), :]
            obuf[s] = ibuf[s] + pos[:, None, :]
            ocopy(i, s).start()

            @pl.when(t + 1 < nsup)
            def _():
                icopy(i + _NB, s).start()

        return carry

    jax.lax.fori_loop(0, nsup, super_body, 0)

    for s in range(_NB):
        ocopy(n - _NB + s, s).wait()


def kernel(x, pos_table):
    B, S, D = x.shape
    xt = jnp.transpose(x, (1, 0, 2))
    out_t = pl.pallas_call(
        _posadd_kernel,
        in_specs=[
            pl.BlockSpec((pos_table.shape[0], D), lambda: (0, 0)),
            pl.BlockSpec(memory_space=pl.ANY),
        ],
        out_specs=pl.BlockSpec(memory_space=pl.ANY),
        out_shape=jax.ShapeDtypeStruct((S, B, D), x.dtype),
        scratch_shapes=[
            pltpu.VMEM((_NB, _CH, B, D), jnp.float32),
            pltpu.VMEM((_NB, _CH, B, D), jnp.float32),
            pltpu.SemaphoreType.DMA((_NB,)),
            pltpu.SemaphoreType.DMA((_NB,)),
        ],
    )(pos_table, xt)
    return jnp.transpose(out_t, (1, 0, 2))


# speedup vs baseline: 1.0089x; 1.0089x over previous
import jax
import jax.numpy as jnp
from jax.experimental import pallas as pl

_SB = 24


def _posadd_kernel(x_ref, pos_ref, o_ref):
    i = pl.program_id(0)
    pos = pos_ref[pl.ds(i * _SB, _SB), :]
    o_ref[...] = x_ref[...] + pos[:, None, :]


def kernel(x, pos_table):
    B, S, D = x.shape
    xt = jnp.transpose(x, (1, 0, 2))
    out_t = pl.pallas_call(
        _posadd_kernel,
        grid=(pl.cdiv(S, _SB),),
        in_specs=[
            pl.BlockSpec((_SB, B, D), lambda i: (i, 0, 0)),
            pl.BlockSpec((512, D), lambda i: (0, 0)),
        ],
        out_specs=pl.BlockSpec((_SB, B, D), lambda i: (i, 0, 0)),
        out_shape=jax.ShapeDtypeStruct((S, B, D), x.dtype),
    )(xt, pos_table)
    return jnp.transpose(out_t, (1, 0, 2))
